# baseline (speedup 1.0000x reference)
import jax
import jax.numpy as jnp
from jax.experimental import pallas as pl

_B = 32
_M = 96
_C = 32
_NF = 320
_K = 5


def _lrelu(v):
    return jnp.where(v >= 0, v, 0.01 * v)


def _fwd(x_ref, L_ref, w1_ref, w2_ref, w3_ref, b1_ref, b2_ref, b3_ref,
         fcw_ref, fcb_ref, o_ref):
    f32 = jnp.float32
    bf16 = jnp.bfloat16

    L = L_ref[0]
    Ts = [jnp.eye(_M, dtype=f32), L]
    for _ in range(2, _K):
        Ts.append(2.0 * jnp.dot(L, Ts[-1], preferred_element_type=f32)
                  - Ts[-2])
    Tcat = jnp.concatenate(Ts[1:], axis=1).astype(bf16)

    def conv_blocks(X0, w_ref, b_ref):
        Ys = [jax.lax.dot_general(
                  X0, w_ref[0, k], (((1,), (1,)), ((), ())),
                  preferred_element_type=f32).astype(bf16)
              for k in range(_K)]
        bias = b_ref[0]
        out = []
        for b in range(_B):
            rows = slice(b * _M, (b + 1) * _M)
            Scat = jnp.concatenate([Y[rows] for Y in Ys[1:]],
                                   axis=0)
            out.append(jnp.dot(Tcat, Scat, preferred_element_type=f32)
                       + Ys[0][rows].astype(f32) + bias)
        return out

    x = x_ref[0].reshape(_B * _M, _C)

    h = jnp.concatenate(
        [_lrelu(z) for z in conv_blocks(x, w1_ref, b1_ref)],
        axis=0).astype(bf16)
    h = jnp.concatenate(
        [_lrelu(z) for z in conv_blocks(h, w2_ref, b2_ref)],
        axis=0).astype(bf16)

    fcw = fcw_ref[...]
    fcb = fcb_ref[...]
    for b, z3 in enumerate(conv_blocks(h, w3_ref, b3_ref)):
        lg = jnp.dot(fcw, z3, preferred_element_type=f32) + fcb
        o_ref[0, b] = jax.nn.sigmoid(lg)


def kernel(L0, L1, L2, x0, x1, x2, D0, D1, D2, adD0, adD1, adD2,
           theta0_1, theta0_2, theta0_3, bias0_1, bias0_2, bias0_3,
           theta1_1, theta1_2, theta1_3, bias1_1, bias1_2, bias1_3,
           theta2_1, theta2_2, theta2_3, bias2_1, bias2_2, bias2_3,
           fc_w, fc_b):
    bf16 = jnp.bfloat16
    xs = jnp.stack([x.astype(bf16).transpose(0, 2, 1) for x in (x0, x1, x2)])
    Ls = jnp.stack([L0, L1, L2])
    def _kof(t):
        o, i, k = t.shape
        return t.astype(bf16).reshape(o * i, k).T.reshape(k, o, i)
    W1 = jnp.stack([_kof(t) for t in (theta0_1, theta1_1, theta2_1)])
    W2 = jnp.stack([_kof(t) for t in (theta0_2, theta1_2, theta2_2)])
    W3 = jnp.stack([_kof(t) for t in (theta0_3, theta1_3, theta2_3)])
    b1 = jnp.stack([b[:, :, 0] for b in (bias0_1, bias1_1, bias2_1)])
    b2 = jnp.stack([b[:, :, 0] for b in (bias0_2, bias1_2, bias2_2)])
    b3 = jnp.stack([b[:, :, 0] for b in (bias0_3, bias1_3, bias2_3)])
    fcb = fc_b.reshape(2, 1)

    out = pl.pallas_call(
        _fwd,
        grid=(3,),
        in_specs=[
            pl.BlockSpec((1, _B, _M, _C), lambda i: (i, 0, 0, 0)),
            pl.BlockSpec((1, _M, _M), lambda i: (i, 0, 0)),
            pl.BlockSpec((1, _K, _NF, _C), lambda i: (i, 0, 0, 0)),
            pl.BlockSpec((1, _K, _NF, _NF), lambda i: (i, 0, 0, 0)),
            pl.BlockSpec((1, _K, _C, _NF), lambda i: (i, 0, 0, 0)),
            pl.BlockSpec((1, 1, _NF), lambda i: (i, 0, 0)),
            pl.BlockSpec((1, 1, _NF), lambda i: (i, 0, 0)),
            pl.BlockSpec((1, 1, _C), lambda i: (i, 0, 0)),
            pl.BlockSpec((2, _M), lambda i: (0, 0)),
            pl.BlockSpec((2, 1), lambda i: (0, 0)),
        ],
        out_specs=pl.BlockSpec((1, _B, 2, _C), lambda i: (i, 0, 0, 0)),
        out_shape=jax.ShapeDtypeStruct((3, _B, 2, _C), jnp.float32),
    )(xs, Ls, W1, W2, W3, b1, b2, b3, fc_w, fcb)

    return out.transpose(1, 0, 3, 2).reshape(_B, 3 * _C, 2)


# speedup vs baseline: 1.0425x; 1.0425x over previous
import jax
import jax.numpy as jnp
from jax.experimental import pallas as pl

_B = 32
_M = 96
_C = 32
_NF = 320
_K = 5


def _lrelu(v):
    return jnp.where(v >= 0, v, 0.01 * v)


def _fwd(*refs):
    o_ref = refs[-1]
    fcw = refs[24][...]
    fcb = refs[25][...]
    f32 = jnp.float32
    bf16 = jnp.bfloat16

    for lvl in range(3):
        (x_ref, L_ref, w1_ref, w2_ref, w3_ref,
         b1_ref, b2_ref, b3_ref) = refs[lvl * 8:lvl * 8 + 8]

        L = L_ref[...]
        Ts = [jnp.eye(_M, dtype=f32), L]
        for _ in range(2, _K):
            Ts.append(2.0 * jnp.dot(L, Ts[-1], preferred_element_type=f32)
                      - Ts[-2])
        Tcat = jnp.concatenate(Ts[1:], axis=1).astype(bf16)

        def conv_blocks(X0, w_ref, b_ref):
            Ys = [jnp.dot(X0, w_ref[k], preferred_element_type=f32)
                  .astype(bf16) for k in range(_K)]
            bias = b_ref[...]
            out = []
            for b in range(_B):
                rows = slice(b * _M, (b + 1) * _M)
                Scat = jnp.concatenate([Y[rows] for Y in Ys[1:]],
                                       axis=0)
                out.append(jnp.dot(Tcat, Scat, preferred_element_type=f32)
                           + Ys[0][rows].astype(f32) + bias)
            return out

        x = x_ref[...].reshape(_B * _M, _C)

        h = jnp.concatenate(
            [_lrelu(z) for z in conv_blocks(x, w1_ref, b1_ref)],
            axis=0).astype(bf16)
        h = jnp.concatenate(
            [_lrelu(z) for z in conv_blocks(h, w2_ref, b2_ref)],
            axis=0).astype(bf16)

        for b, z3 in enumerate(conv_blocks(h, w3_ref, b3_ref)):
            lgT = jax.lax.dot_general(
                z3, fcw, (((0,), (1,)), ((), ())),
                preferred_element_type=f32) + fcb
            o_ref[b, lvl * _C:(lvl + 1) * _C, :] = jax.nn.sigmoid(lgT)


def kernel(L0, L1, L2, x0, x1, x2, D0, D1, D2, adD0, adD1, adD2,
           theta0_1, theta0_2, theta0_3, bias0_1, bias0_2, bias0_3,
           theta1_1, theta1_2, theta1_3, bias1_1, bias1_2, bias1_3,
           theta2_1, theta2_2, theta2_3, bias2_1, bias2_2, bias2_3,
           fc_w, fc_b):
    bf16 = jnp.bfloat16

    def prep_x(x):
        return x.astype(bf16).transpose(0, 2, 1)

    def prep_w(t):
        return t.astype(bf16).transpose(2, 1, 0)

    def prep_b(b):
        return b[:, :, 0]

    args = []
    for x, L, t1, t2, t3, b1, b2, b3 in (
            (x0, L0, theta0_1, theta0_2, theta0_3, bias0_1, bias0_2, bias0_3),
            (x1, L1, theta1_1, theta1_2, theta1_3, bias1_1, bias1_2, bias1_3),
            (x2, L2, theta2_1, theta2_2, theta2_3, bias2_1, bias2_2, bias2_3)):
        args += [prep_x(x), L, prep_w(t1), prep_w(t2), prep_w(t3),
                 prep_b(b1), prep_b(b2), prep_b(b3)]
    args += [fc_w, fc_b.reshape(1, 2)]

    return pl.pallas_call(
        _fwd,
        out_shape=jax.ShapeDtypeStruct((_B, 3 * _C, 2), jnp.float32),
    )(*args)
